# baseline (speedup 1.0000x reference)
import functools

import jax
import jax.numpy as jnp
from jax.experimental import pallas as pl
from jax.experimental.pallas import tpu as pltpu

_HUFFMAN_PATHS = ((0, 1), (1, 0), (0, 0, 1), (1, 1, 0))
_NCOL = 16


def _body(h_ref, tw_ref, w_ref, out_ref):
    h = h_ref[...]
    bm = h.shape[0]
    tw = tw_ref[...]
    n = pl.num_programs(0) * bm
    x = jnp.dot(h, w_ref[...], preferred_element_type=jnp.float32)
    soft = jnp.maximum(x, 0.0) + jnp.log1p(jnp.exp(-jnp.abs(x)))
    total = jnp.float32(0.0)
    off = 0
    for w, path in enumerate(_HUFFMAN_PATHS):
        lw = len(path)
        ones = [off + j for j, b in enumerate(path) if b == 1]
        lo, hi = ones[0], ones[-1] + 1
        soft_w = jnp.sum(soft[:, off : off + lw], axis=1, keepdims=True)
        xs_w = jnp.sum(x[:, lo:hi], axis=1, keepdims=True)
        per_row = (soft_w - xs_w) * (1.0 / lw)
        sel = (tw == w).astype(jnp.float32)
        total = total + jnp.sum(sel * per_row)
        off += lw

    @pl.when(pl.program_id(0) == 0)
    def _():
        out_ref[0, 0] = 0.0

    out_ref[0, 0] += total / jnp.float32(n)


@functools.partial(jax.jit, static_argnames=("interpret", "bm"))
def kernel(hidden, target_words, W_0, W_1, W_2, W_3, interpret=False, bm=512):
    batch, hdim = hidden.shape
    grid = batch // bm

    wstack = jnp.concatenate([W_0, W_1, W_2, W_3], axis=0)
    wstack = jnp.pad(wstack, ((0, _NCOL - wstack.shape[0]), (0, 0))).T
    tw2d = target_words.astype(jnp.int32).reshape(batch, 1)

    out = pl.pallas_call(
        _body,
        grid=(grid,),
        in_specs=[
            pl.BlockSpec((bm, hdim), lambda i: (i, 0)),
            pl.BlockSpec((bm, 1), lambda i: (i, 0)),
            pl.BlockSpec((hdim, _NCOL), lambda i: (0, 0)),
        ],
        out_specs=pl.BlockSpec(
            (1, 1), lambda i: (0, 0), memory_space=pltpu.SMEM
        ),
        out_shape=jax.ShapeDtypeStruct((1, 1), jnp.float32),
        interpret=interpret,
    )(hidden, tw2d, wstack)
    return out[0, 0]


# speedup vs baseline: 2.5394x; 2.5394x over previous
import functools

import jax
import jax.numpy as jnp
from jax.experimental import pallas as pl
from jax.experimental.pallas import tpu as pltpu

_HUFFMAN_PATHS = ((0, 1), (1, 0), (0, 0, 1), (1, 1, 0))
_NCOL = 16


def _coeff_tables(n):
    r = jax.lax.broadcasted_iota(jnp.int32, (8, _NCOL), 0)
    c = jax.lax.broadcasted_iota(jnp.int32, (8, _NCOL), 1)
    a = jnp.zeros((8, _NCOL), jnp.float32)
    b = jnp.zeros((8, _NCOL), jnp.float32)
    off = 0
    for w, path in enumerate(_HUFFMAN_PATHS):
        lw = len(path)
        coeff = 1.0 / (lw * n)
        in_word = (r == w) & (c >= off) & (c < off + lw)
        a = jnp.where(in_word, coeff, a)
        ones = [off + j for j, bit in enumerate(path) if bit == 1]
        lo, hi = ones[0], ones[-1] + 1
        b = jnp.where((r == w) & (c >= lo) & (c < hi), coeff, b)
        off += lw
    return a, b


def _body(h_ref, tw_ref, w0_ref, w1_ref, w2_ref, w3_ref, out_ref, wt_ref):
    bm = h_ref.shape[0]
    n = pl.num_programs(0) * bm

    @pl.when(pl.program_id(0) == 0)
    def _():
        wt_ref[:, 0:2] = w0_ref[...].T
        wt_ref[:, 2:4] = w1_ref[...].T
        wt_ref[:, 4:7] = w2_ref[...].T
        wt_ref[:, 7:10] = w3_ref[...].T
        wt_ref[:, 10:16] = jnp.zeros((512, 6), jnp.float32)
        out_ref[0, 0] = 0.0

    h = h_ref[...]
    tw = tw_ref[...]
    x = jnp.dot(h, wt_ref[...], preferred_element_type=jnp.float32)
    soft = jnp.maximum(x, 0.0) + jnp.log1p(jnp.exp(-jnp.abs(x)))
    onehot = (tw == jax.lax.broadcasted_iota(jnp.int32, (bm, 8), 1)).astype(
        jnp.float32
    )
    a_tab, b_tab = _coeff_tables(n)
    mask = jnp.dot(onehot, a_tab, preferred_element_type=jnp.float32)
    bsel = jnp.dot(onehot, b_tab, preferred_element_type=jnp.float32)
    out_ref[0, 0] += jnp.sum(mask * soft) - jnp.sum(bsel * x)


@functools.partial(jax.jit, static_argnames=("interpret", "bm"))
def kernel(hidden, target_words, W_0, W_1, W_2, W_3, interpret=False, bm=512):
    batch, hdim = hidden.shape
    grid = batch // bm
    tw2d = target_words.astype(jnp.int32).reshape(batch, 1)

    full = lambda shape: pl.BlockSpec(shape, lambda i: (0, 0))
    out = pl.pallas_call(
        _body,
        grid=(grid,),
        in_specs=[
            pl.BlockSpec((bm, hdim), lambda i: (i, 0)),
            pl.BlockSpec((bm, 1), lambda i: (i, 0)),
            full(W_0.shape),
            full(W_1.shape),
            full(W_2.shape),
            full(W_3.shape),
        ],
        out_specs=pl.BlockSpec(
            (1, 1), lambda i: (0, 0), memory_space=pltpu.SMEM
        ),
        out_shape=jax.ShapeDtypeStruct((1, 1), jnp.float32),
        scratch_shapes=[pltpu.VMEM((hdim, _NCOL), jnp.float32)],
        interpret=interpret,
    )(hidden, tw2d, W_0, W_1, W_2, W_3)
    return out[0, 0]


# speedup vs baseline: 3.1006x; 1.2210x over previous
import functools

import jax
import jax.numpy as jnp
from jax.experimental import pallas as pl
from jax.experimental.pallas import tpu as pltpu

_HUFFMAN_PATHS = ((0, 1), (1, 0), (0, 0, 1), (1, 1, 0))
_NCOL = 16


def _coeff_tables(n):
    r = jax.lax.broadcasted_iota(jnp.int32, (8, _NCOL), 0)
    c = jax.lax.broadcasted_iota(jnp.int32, (8, _NCOL), 1)
    a = jnp.zeros((8, _NCOL), jnp.float32)
    b = jnp.zeros((8, _NCOL), jnp.float32)
    off = 0
    for w, path in enumerate(_HUFFMAN_PATHS):
        lw = len(path)
        coeff = 1.0 / (lw * n)
        in_word = (r == w) & (c >= off) & (c < off + lw)
        a = jnp.where(in_word, coeff, a)
        ones = [off + j for j, bit in enumerate(path) if bit == 1]
        lo, hi = ones[0], ones[-1] + 1
        b = jnp.where((r == w) & (c >= lo) & (c < hi), coeff, b)
        off += lw
    return a, b


def _body(h_ref, tw_ref, w0_ref, w1_ref, w2_ref, w3_ref, out_ref, wt_ref):
    bm = h_ref.shape[0]
    n = pl.num_programs(0) * bm

    @pl.when(pl.program_id(0) == 0)
    def _():
        wt_ref[:, 0:2] = w0_ref[...].T
        wt_ref[:, 2:4] = w1_ref[...].T
        wt_ref[:, 4:7] = w2_ref[...].T
        wt_ref[:, 7:10] = w3_ref[...].T
        wt_ref[:, 10:16] = jnp.zeros((512, 6), jnp.float32)
        out_ref[0, 0] = 0.0

    h = h_ref[...]
    tw = tw_ref[...]
    x = jnp.dot(h, wt_ref[...], preferred_element_type=jnp.float32)
    soft = jnp.maximum(x, 0.0) + jnp.log1p(jnp.exp(-jnp.abs(x)))
    onehot = (tw == jax.lax.broadcasted_iota(jnp.int32, (bm, 8), 1)).astype(
        jnp.float32
    )
    a_tab, b_tab = _coeff_tables(n)
    mask = jnp.dot(onehot, a_tab, preferred_element_type=jnp.float32)
    bsel = jnp.dot(onehot, b_tab, preferred_element_type=jnp.float32)
    out_ref[0, 0] += jnp.sum(mask * soft) - jnp.sum(bsel * x)


@functools.partial(jax.jit, static_argnames=("interpret", "bm"))
def kernel(hidden, target_words, W_0, W_1, W_2, W_3, interpret=False, bm=1024):
    batch, hdim = hidden.shape
    grid = batch // bm
    tw2d = target_words.astype(jnp.int32).reshape(batch, 1)

    full = lambda shape: pl.BlockSpec(shape, lambda i: (0, 0))
    out = pl.pallas_call(
        _body,
        grid=(grid,),
        in_specs=[
            pl.BlockSpec((bm, hdim), lambda i: (i, 0)),
            pl.BlockSpec((bm, 1), lambda i: (i, 0)),
            full(W_0.shape),
            full(W_1.shape),
            full(W_2.shape),
            full(W_3.shape),
        ],
        out_specs=pl.BlockSpec(
            (1, 1), lambda i: (0, 0), memory_space=pltpu.SMEM
        ),
        out_shape=jax.ShapeDtypeStruct((1, 1), jnp.float32),
        scratch_shapes=[pltpu.VMEM((hdim, _NCOL), jnp.float32)],
        interpret=interpret,
    )(hidden, tw2d, W_0, W_1, W_2, W_3)
    return out[0, 0]


# speedup vs baseline: 3.3612x; 1.0840x over previous
import functools

import jax
import jax.numpy as jnp
from jax.experimental import pallas as pl
from jax.experimental.pallas import tpu as pltpu

_HUFFMAN_PATHS = ((0, 1), (1, 0), (0, 0, 1), (1, 1, 0))
_NCOL = 16


def _coeff_tables(n):
    r = jax.lax.broadcasted_iota(jnp.int32, (8, _NCOL), 0)
    c = jax.lax.broadcasted_iota(jnp.int32, (8, _NCOL), 1)
    a = jnp.zeros((8, _NCOL), jnp.float32)
    b = jnp.zeros((8, _NCOL), jnp.float32)
    off = 0
    for w, path in enumerate(_HUFFMAN_PATHS):
        lw = len(path)
        coeff = 1.0 / (lw * n)
        in_word = (r == w) & (c >= off) & (c < off + lw)
        a = jnp.where(in_word, coeff, a)
        ones = [off + j for j, bit in enumerate(path) if bit == 1]
        lo, hi = ones[0], ones[-1] + 1
        b = jnp.where((r == w) & (c >= lo) & (c < hi), coeff, b)
        off += lw
    return a, b


def _body(h_ref, tw_ref, w0_ref, w1_ref, w2_ref, w3_ref, out_ref, wt_ref):
    bm = h_ref.shape[0]
    n = pl.num_programs(0) * bm

    @pl.when(pl.program_id(0) == 0)
    def _():
        wt_ref[:, 0:2] = w0_ref[...].T
        wt_ref[:, 2:4] = w1_ref[...].T
        wt_ref[:, 4:7] = w2_ref[...].T
        wt_ref[:, 7:10] = w3_ref[...].T
        wt_ref[:, 10:16] = jnp.zeros((512, 6), jnp.float32)
        out_ref[0, 0] = 0.0

    h = h_ref[...]
    tw = tw_ref[...]
    x = jnp.dot(h, wt_ref[...], preferred_element_type=jnp.float32)
    soft = jnp.maximum(x, 0.0) + jnp.log1p(jnp.exp(-jnp.abs(x)))
    onehot = (tw == jax.lax.broadcasted_iota(jnp.int32, (bm, 8), 1)).astype(
        jnp.float32
    )
    a_tab, b_tab = _coeff_tables(n)
    mask = jnp.dot(onehot, a_tab, preferred_element_type=jnp.float32)
    bsel = jnp.dot(onehot, b_tab, preferred_element_type=jnp.float32)
    out_ref[0, 0] += jnp.sum(mask * soft) - jnp.sum(bsel * x)


@functools.partial(jax.jit, static_argnames=("interpret", "bm"))
def kernel(hidden, target_words, W_0, W_1, W_2, W_3, interpret=False, bm=2048):
    batch, hdim = hidden.shape
    grid = batch // bm
    tw2d = target_words.astype(jnp.int32).reshape(batch, 1)

    full = lambda shape: pl.BlockSpec(shape, lambda i: (0, 0))
    out = pl.pallas_call(
        _body,
        grid=(grid,),
        in_specs=[
            pl.BlockSpec((bm, hdim), lambda i: (i, 0)),
            pl.BlockSpec((bm, 1), lambda i: (i, 0)),
            full(W_0.shape),
            full(W_1.shape),
            full(W_2.shape),
            full(W_3.shape),
        ],
        out_specs=pl.BlockSpec(
            (1, 1), lambda i: (0, 0), memory_space=pltpu.SMEM
        ),
        out_shape=jax.ShapeDtypeStruct((1, 1), jnp.float32),
        scratch_shapes=[pltpu.VMEM((hdim, _NCOL), jnp.float32)],
        interpret=interpret,
    )(hidden, tw2d, W_0, W_1, W_2, W_3)
    return out[0, 0]
